# baseline (speedup 1.0000x reference)
import functools

import jax
import jax.numpy as jnp
from jax import lax
from jax.experimental import pallas as pl
from jax.experimental.pallas import tpu as pltpu
from jax.experimental.pallas import tpu_sc as plsc

_E = 64
_K = 2
_T = 1024


def _logits_body(x1_ref, x2_ref, w_ref, b_ref, out_ref):
    nt = (((1,), (1,)), ((), ()))
    h = x1_ref.shape[1]
    acc = lax.dot_general(w_ref[:, :h], x1_ref[...], nt,
                          preferred_element_type=jnp.float32)
    acc += lax.dot_general(w_ref[:, h:], x2_ref[...], nt,
                           preferred_element_type=jnp.float32)
    out_ref[...] = acc + b_ref[...]


def _tc_logits(x, W, b2d, ctok, c):
    ntok, H = x.shape
    grid = ctok // _T
    base = c * grid
    return pl.pallas_call(
        _logits_body,
        grid=(grid,),
        in_specs=[
            pl.BlockSpec((_T, H // 2), lambda i: (base + i, 0)),
            pl.BlockSpec((_T, H // 2), lambda i: (base + i, 1)),
            pl.BlockSpec((_E, H), lambda i: (0, 0)),
            pl.BlockSpec((_E, 1), lambda i: (0, 0)),
        ],
        out_specs=pl.BlockSpec((_E, _T), lambda i: (0, i)),
        out_shape=jax.ShapeDtypeStruct((_E, ctok), jnp.float32),
    )(x, x, W, b2d)


@functools.cache
def _sc_router(ntok):
    info = plsc.get_sparse_core_info()
    nc, ns, L = info.num_cores, info.num_subcores, info.num_lanes
    nw = nc * ns
    tpw = ntok // nw
    mesh = plsc.VectorSubcoreMesh(core_axis_name="c", subcore_axis_name="s")

    @functools.partial(
        pl.kernel,
        out_type=(
            jax.ShapeDtypeStruct((_K, ntok), jnp.int32),
            jax.ShapeDtypeStruct((_K, ntok), jnp.float32),
        ),
        mesh=mesh,
        scratch_types=[
            pltpu.VMEM((_E, tpw), jnp.float32),
            pltpu.VMEM((_K, tpw), jnp.int32),
            pltpu.VMEM((_K, tpw), jnp.float32),
        ],
        compiler_params=pltpu.CompilerParams(use_tc_tiling_on_sc=True),
    )
    def route(logits_hbm, idx_hbm, w_hbm, chunk_v, idx_v, w_v):
        wid = lax.axis_index("s") * nc + lax.axis_index("c")
        base = wid * tpw
        pltpu.sync_copy(logits_hbm.at[:, pl.ds(base, tpw)], chunk_v)

        n_ilp = 4

        def group(g, carry):
            offs = [(g * n_ilp + j) * L for j in range(n_ilp)]
            neg = jnp.full((L,), -jnp.inf, jnp.float32)
            zero = jnp.zeros((L,), jnp.int32)
            m1 = [neg] * n_ilp
            m2 = [neg] * n_ilp
            i1 = [zero] * n_ilp
            i2 = [zero] * n_ilp
            for e in range(_E):
                ev = jnp.full((L,), e, jnp.int32)
                for j in range(n_ilp):
                    v = chunk_v[e, pl.ds(offs[j], L)]
                    gt1 = v > m1[j]
                    gt2 = v > m2[j]
                    m2[j] = jnp.where(gt1, m1[j], jnp.where(gt2, v, m2[j]))
                    i2[j] = jnp.where(gt1, i1[j], jnp.where(gt2, ev, i2[j]))
                    m1[j] = jnp.where(gt1, v, m1[j])
                    i1[j] = jnp.where(gt1, ev, i1[j])
            for j in range(n_ilp):
                w1 = 1.0 / (1.0 + jnp.exp(m2[j] - m1[j]))
                idx_v[0, pl.ds(offs[j], L)] = i1[j]
                idx_v[1, pl.ds(offs[j], L)] = i2[j]
                w_v[0, pl.ds(offs[j], L)] = w1
                w_v[1, pl.ds(offs[j], L)] = 1.0 - w1
            return carry

        lax.fori_loop(0, tpw // (L * n_ilp), group, 0)
        pltpu.sync_copy(idx_v, idx_hbm.at[:, pl.ds(base, tpw)])
        pltpu.sync_copy(w_v, w_hbm.at[:, pl.ds(base, tpw)])

    return route


def kernel(hidden_states, W, b):
    B, S, H = hidden_states.shape
    ntok = B * S
    nchunks = 1
    ctok = ntok // nchunks
    x = hidden_states.reshape(ntok, H)
    b2d = b.reshape(_E, 1)
    router = _sc_router(ctok)
    parts = []
    for c in range(nchunks):
        logits_t = _tc_logits(x, W, b2d, ctok, c)
        parts.append(router(logits_t))
    idx_t = jnp.concatenate([p[0] for p in parts], axis=1)
    w_t = jnp.concatenate([p[1] for p in parts], axis=1)
    expert_indices = idx_t.T.reshape(B, S, _K)
    routing_weights = w_t.T.reshape(B, S, _K)
    return expert_indices, routing_weights


# speedup vs baseline: 1.0025x; 1.0025x over previous
import functools

import jax
import jax.numpy as jnp
from jax import lax
from jax.experimental import pallas as pl
from jax.experimental.pallas import tpu as pltpu
from jax.experimental.pallas import tpu_sc as plsc

_E = 64
_K = 2
_T = 1024


def _logits_body(x_ref, w_ref, b_ref, out_ref):
    acc = lax.dot_general(
        w_ref[...], x_ref[...],
        (((1,), (1,)), ((), ())),
        preferred_element_type=jnp.float32,
    )
    out_ref[...] = acc + b_ref[...]


def _tc_logits(x, W, b2d, ctok, c):
    ntok, H = x.shape
    grid = ctok // _T
    base = c * grid
    return pl.pallas_call(
        _logits_body,
        grid=(grid,),
        in_specs=[
            pl.BlockSpec((_T, H), lambda i: (base + i, 0)),
            pl.BlockSpec((_E, H), lambda i: (0, 0)),
            pl.BlockSpec((_E, 1), lambda i: (0, 0)),
        ],
        out_specs=pl.BlockSpec((_E, _T), lambda i: (0, i)),
        out_shape=jax.ShapeDtypeStruct((_E, ctok), jnp.float32),
    )(x, W, b2d)


@functools.cache
def _sc_router(ntok):
    info = plsc.get_sparse_core_info()
    nc, ns, L = info.num_cores, info.num_subcores, info.num_lanes
    nw = nc * ns
    tpw = ntok // nw
    mesh = plsc.VectorSubcoreMesh(core_axis_name="c", subcore_axis_name="s")

    @functools.partial(
        pl.kernel,
        out_type=(
            jax.ShapeDtypeStruct((_K, ntok), jnp.int32),
            jax.ShapeDtypeStruct((_K, ntok), jnp.float32),
        ),
        mesh=mesh,
        scratch_types=[
            pltpu.VMEM((_E, tpw), jnp.float32),
            pltpu.VMEM((_K, tpw), jnp.int32),
            pltpu.VMEM((_K, tpw), jnp.float32),
        ],
        compiler_params=pltpu.CompilerParams(
            use_tc_tiling_on_sc=True, skip_device_barrier=True),
    )
    def route(logits_hbm, idx_hbm, w_hbm, chunk_v, idx_v, w_v):
        wid = lax.axis_index("s") * nc + lax.axis_index("c")
        base = wid * tpw
        pltpu.sync_copy(logits_hbm.at[:, pl.ds(base, tpw)], chunk_v)

        n_ilp = 4

        def group(g, carry):
            offs = [(g * n_ilp + j) * L for j in range(n_ilp)]
            neg = jnp.full((L,), -jnp.inf, jnp.float32)
            zero = jnp.zeros((L,), jnp.int32)
            m1 = [neg] * n_ilp
            m2 = [neg] * n_ilp
            i1 = [zero] * n_ilp
            i2 = [zero] * n_ilp
            for e in range(_E):
                ev = jnp.full((L,), e, jnp.int32)
                for j in range(n_ilp):
                    v = chunk_v[e, pl.ds(offs[j], L)]
                    gt1 = v > m1[j]
                    gt2 = v > m2[j]
                    m2[j] = jnp.where(gt1, m1[j], jnp.where(gt2, v, m2[j]))
                    i2[j] = jnp.where(gt1, i1[j], jnp.where(gt2, ev, i2[j]))
                    m1[j] = jnp.where(gt1, v, m1[j])
                    i1[j] = jnp.where(gt1, ev, i1[j])
            for j in range(n_ilp):
                w1 = 1.0 / (1.0 + jnp.exp(m2[j] - m1[j]))
                idx_v[0, pl.ds(offs[j], L)] = i1[j]
                idx_v[1, pl.ds(offs[j], L)] = i2[j]
                w_v[0, pl.ds(offs[j], L)] = w1
                w_v[1, pl.ds(offs[j], L)] = 1.0 - w1
            return carry

        lax.fori_loop(0, tpw // (L * n_ilp), group, 0)
        pltpu.sync_copy(idx_v, idx_hbm.at[:, pl.ds(base, tpw)])
        pltpu.sync_copy(w_v, w_hbm.at[:, pl.ds(base, tpw)])

    return route


def kernel(hidden_states, W, b):
    B, S, H = hidden_states.shape
    ntok = B * S
    nchunks = 1
    ctok = ntok // nchunks
    x = hidden_states.reshape(ntok, H)
    b2d = b.reshape(_E, 1)
    router = _sc_router(ctok)
    parts = []
    for c in range(nchunks):
        logits_t = _tc_logits(x, W, b2d, ctok, c)
        parts.append(router(logits_t))
    idx_t = jnp.concatenate([p[0] for p in parts], axis=1)
    w_t = jnp.concatenate([p[1] for p in parts], axis=1)
    expert_indices = idx_t.T.reshape(B, S, _K)
    routing_weights = w_t.T.reshape(B, S, _K)
    return expert_indices, routing_weights


# speedup vs baseline: 1.0346x; 1.0320x over previous
import functools

import jax
import jax.numpy as jnp
from jax import lax
from jax.experimental import pallas as pl
from jax.experimental.pallas import tpu as pltpu
from jax.experimental.pallas import tpu_sc as plsc

_E = 64
_K = 2
_T = 1024


def _logits_body(x_ref, w_ref, b_ref, out_ref):
    acc = lax.dot_general(
        w_ref[...], x_ref[...],
        (((1,), (1,)), ((), ())),
        preferred_element_type=jnp.float32,
    )
    out_ref[...] = acc + b_ref[...][:, None]


def _tc_logits(x, W, b, ctok, c):
    ntok, H = x.shape
    grid = ctok // _T
    base = c * grid
    return pl.pallas_call(
        _logits_body,
        grid=(grid,),
        in_specs=[
            pl.BlockSpec((_T, H), lambda i: (base + i, 0)),
            pl.BlockSpec((_E, H), lambda i: (0, 0)),
            pl.BlockSpec((_E,), lambda i: (0,)),
        ],
        out_specs=pl.BlockSpec((_E, _T), lambda i: (0, i)),
        out_shape=jax.ShapeDtypeStruct((_E, ctok), jnp.float32),
    )(x, W, b)


@functools.cache
def _sc_router(ntok):
    info = plsc.get_sparse_core_info()
    nc, ns, L = info.num_cores, info.num_subcores, info.num_lanes
    nw = nc * ns
    tpw = ntok // nw
    mesh = plsc.VectorSubcoreMesh(core_axis_name="c", subcore_axis_name="s")

    @functools.partial(
        pl.kernel,
        out_type=(
            jax.ShapeDtypeStruct((_K, ntok), jnp.int32),
            jax.ShapeDtypeStruct((_K, ntok), jnp.float32),
        ),
        mesh=mesh,
        scratch_types=[
            pltpu.VMEM((_E, tpw), jnp.float32),
            pltpu.VMEM((_K, tpw), jnp.int32),
            pltpu.VMEM((_K, tpw), jnp.float32),
        ],
        compiler_params=pltpu.CompilerParams(use_tc_tiling_on_sc=True),
    )
    def route(logits_hbm, idx_hbm, w_hbm, chunk_v, idx_v, w_v):
        wid = lax.axis_index("s") * nc + lax.axis_index("c")
        base = wid * tpw
        pltpu.sync_copy(logits_hbm.at[:, pl.ds(base, tpw)], chunk_v)

        n_ilp = 2

        def group(g, carry):
            offs = [(g * n_ilp + j) * L for j in range(n_ilp)]
            neg = jnp.full((L,), -jnp.inf, jnp.float32)
            zero = jnp.zeros((L,), jnp.int32)
            m1 = [neg] * n_ilp
            m2 = [neg] * n_ilp
            i1 = [zero] * n_ilp
            i2 = [zero] * n_ilp
            for e in range(_E):
                ev = jnp.full((L,), e, jnp.int32)
                for j in range(n_ilp):
                    v = chunk_v[e, pl.ds(offs[j], L)]
                    gt1 = v > m1[j]
                    gt2 = v > m2[j]
                    m2[j] = jnp.where(gt1, m1[j], jnp.where(gt2, v, m2[j]))
                    i2[j] = jnp.where(gt1, i1[j], jnp.where(gt2, ev, i2[j]))
                    m1[j] = jnp.where(gt1, v, m1[j])
                    i1[j] = jnp.where(gt1, ev, i1[j])
            for j in range(n_ilp):
                w1 = 1.0 / (1.0 + jnp.exp(m2[j] - m1[j]))
                idx_v[0, pl.ds(offs[j], L)] = i1[j]
                idx_v[1, pl.ds(offs[j], L)] = i2[j]
                w_v[0, pl.ds(offs[j], L)] = w1
                w_v[1, pl.ds(offs[j], L)] = 1.0 - w1
            return carry

        lax.fori_loop(0, tpw // (L * n_ilp), group, 0)
        pltpu.sync_copy(idx_v, idx_hbm.at[:, pl.ds(base, tpw)])
        pltpu.sync_copy(w_v, w_hbm.at[:, pl.ds(base, tpw)])

    return route


def kernel(hidden_states, W, b):
    B, S, H = hidden_states.shape
    ntok = B * S
    nchunks = 1
    ctok = ntok // nchunks
    x = hidden_states.reshape(ntok, H)
    router = _sc_router(ctok)
    parts = []
    for c in range(nchunks):
        logits_t = _tc_logits(x, W, b, ctok, c)
        parts.append(router(logits_t))
    idx_t = jnp.concatenate([p[0] for p in parts], axis=1)
    w_t = jnp.concatenate([p[1] for p in parts], axis=1)
    expert_indices = idx_t.T.reshape(B, S, _K)
    routing_weights = w_t.T.reshape(B, S, _K)
    return expert_indices, routing_weights
